# baseline (speedup 1.0000x reference)
import functools
import jax
import jax.numpy as jnp
from jax import lax
from jax.experimental import pallas as pl
from jax.experimental.pallas import tpu as pltpu
from jax.experimental.pallas import tpu_sc as plsc

K = 512
D = 32
DA = 40
TW = 48
DECAY = 0.9
EPS = 1e-5

NC = 2
NS = 16
NW = NC * NS
CH = 1024


def _argmin_body(T, NB, x_ref, emb_ref, idx_ref, sx2_ref, embA_ref, acc_ref):
    i = pl.program_id(0)

    @pl.when(i == 0)
    def _():
        E = emb_ref[...]
        e2 = jnp.sum(E * E, axis=1, keepdims=True)
        embA_ref[...] = jnp.concatenate(
            [-2.0 * E, e2, jnp.zeros((K, DA - D - 1), jnp.float32)], axis=1)
        acc_ref[0] = 0.0

    x = x_ref[...]
    xa = jnp.concatenate([x, jnp.ones((T, DA - D), jnp.float32)], axis=1)
    dist2 = lax.dot_general(embA_ref[...], xa, (((1,), (1,)), ((), ())),
                            preferred_element_type=jnp.float32)
    mind = jnp.min(dist2, axis=0, keepdims=True)
    fiota = lax.broadcasted_iota(jnp.int32, (K, T), 0).astype(jnp.float32)
    cand = jnp.where(dist2 == mind, fiota, float(K))
    idxf = jnp.min(cand, axis=0, keepdims=True)
    idx_ref[0, i % 8, :] = idxf[0, :].astype(jnp.int32)
    acc_ref[0] += jnp.sum(x * x)

    @pl.when(i == NB - 1)
    def _():
        sx2_ref[...] = jnp.reshape(acc_ref[0], (1, 1))


def _tc_argmin(flat, embedding, T=4096):
    N = flat.shape[0]
    NB = N // T
    idx, sx2 = pl.pallas_call(
        functools.partial(_argmin_body, T, NB),
        grid=(NB,),
        in_specs=[
            pl.BlockSpec((T, D), lambda i: (i, 0)),
            pl.BlockSpec((K, D), lambda i: (0, 0)),
        ],
        out_specs=[
            pl.BlockSpec((1, 8, T), lambda i: (i // 8, 0, 0)),
            pl.BlockSpec((1, 1), lambda i: (0, 0)),
        ],
        out_shape=[
            jax.ShapeDtypeStruct((NB // 8, 8, T), jnp.int32),
            jax.ShapeDtypeStruct((1, 1), jnp.float32),
        ],
        scratch_shapes=[
            pltpu.VMEM((K, DA), jnp.float32),
            pltpu.SMEM((1,), jnp.float32),
        ],
    )(flat, embedding)
    return idx.reshape(N), sx2


def _sc_body(NPW, xflat_hbm, idx_hbm, out_hbm, tab, tb2, xbuf, ibuf):
    c = lax.axis_index("c")
    s = lax.axis_index("s")
    wid = c * NS + s
    tok0 = wid * NPW

    zeros16 = jnp.zeros((16,), jnp.float32)
    iota16 = lax.broadcasted_iota(jnp.int32, (16,), 0)
    cnt16 = jnp.where(iota16 == 0, 1.0, 0.0).astype(jnp.float32)

    def zero_body(j, _):
        tab[pl.ds(j * 16, 16)] = zeros16
        tb2[pl.ds(j * 16, 16)] = zeros16
        return 0
    lax.fori_loop(0, K * TW // 16, zero_body, 0)

    def chunk_body(ci, _):
        base = tok0 + ci * CH
        pltpu.sync_copy(xflat_hbm.at[pl.ds(base * D, CH * D)], xbuf)
        pltpu.sync_copy(idx_hbm.at[pl.ds(base, CH)], ibuf)

        def grp_body(g, _):
            iv = ibuf[pl.ds(g * 16, 16)]
            for j in range(16):
                t = g * 16 + j
                dst = tab if j % 2 == 0 else tb2
                o = iv[j] * TW
                xlo = xbuf[pl.ds(t * D, 16)]
                xhi = xbuf[pl.ds(t * D + 16, 16)]
                plsc.addupdate(dst.at[pl.ds(o, 16)], xlo)
                plsc.addupdate(dst.at[pl.ds(o + 16, 16)], xhi)
                plsc.addupdate(dst.at[pl.ds(o + 32, 16)], cnt16)
            return 0
        lax.fori_loop(0, CH // 16, grp_body, 0)
        return 0
    lax.fori_loop(0, NPW // CH, chunk_body, 0)

    def merge_body(j, _):
        tab[pl.ds(j * 16, 16)] += tb2[pl.ds(j * 16, 16)]
        return 0
    lax.fori_loop(0, K * TW // 16, merge_body, 0)
    pltpu.sync_copy(tab, out_hbm.at[pl.ds(wid * (K * TW), K * TW)])


def _sc_scatter(xflat, idx):
    N = idx.shape[0]
    NPW = N // NW
    mesh = plsc.VectorSubcoreMesh(core_axis_name="c", subcore_axis_name="s",
                                  num_cores=NC, num_subcores=NS)
    f = pl.kernel(
        functools.partial(_sc_body, NPW),
        out_type=jax.ShapeDtypeStruct((NW * K * TW,), jnp.float32),
        mesh=mesh,
        scratch_types=[
            pltpu.VMEM((K * TW,), jnp.float32),
            pltpu.VMEM((K * TW,), jnp.float32),
            pltpu.VMEM((CH * D,), jnp.float32),
            pltpu.VMEM((CH,), jnp.int32),
        ],
    )
    return f(xflat, idx).reshape(NW, K, TW)


def _final_body(N, part_ref, emaw_ref, cs_ref, sx2_ref, out_ref, acc_ref):
    i = pl.program_id(0)

    @pl.when(i == 0)
    def _():
        acc_ref[...] = jnp.zeros_like(acc_ref)

    acc_ref[...] += part_ref[0]

    @pl.when(i == NW - 1)
    def _():
        counts = acc_ref[:, D:D + 1]
        dw = acc_ref[:, 0:D]
        cs = cs_ref[...] * DECAY + (1.0 - DECAY) * counts
        n = jnp.sum(cs)
        csn = (cs + EPS) / (n + K * EPS) * n
        ema_w_new = emaw_ref[...] * DECAY + (1.0 - DECAY) * dw
        e_new = ema_w_new / csn
        s1 = jnp.sum(dw * e_new)
        s2 = jnp.sum(counts * jnp.sum(e_new * e_new, axis=1, keepdims=True))
        loss = (sx2_ref[0, 0] - 2.0 * s1 + s2) / (N * D)
        out_ref[...] = jnp.reshape(loss, (1, 1))


def _tc_final(parts, ema_w, cs, sx2, N):
    return pl.pallas_call(
        functools.partial(_final_body, N),
        grid=(NW,),
        in_specs=[
            pl.BlockSpec((1, K, TW), lambda i: (i, 0, 0)),
            pl.BlockSpec((K, D), lambda i: (0, 0)),
            pl.BlockSpec((K, 1), lambda i: (0, 0)),
            pl.BlockSpec((1, 1), lambda i: (0, 0)),
        ],
        out_specs=pl.BlockSpec((1, 1), lambda i: (0, 0)),
        out_shape=jax.ShapeDtypeStruct((1, 1), jnp.float32),
        scratch_shapes=[pltpu.VMEM((K, TW), jnp.float32)],
    )(parts, ema_w, cs, sx2)


def kernel(inputs, embedding, ema_w, ema_cluster_size):
    N = inputs.shape[0] * inputs.shape[1]
    flat = inputs.reshape(N, D)
    idx, sx2 = _tc_argmin(flat, embedding)
    parts = _sc_scatter(flat.reshape(N * D), idx)
    loss = _tc_final(parts, ema_w, ema_cluster_size.reshape(K, 1), sx2, N)
    return loss[0, 0]


# speedup vs baseline: 1.0693x; 1.0693x over previous
import functools
import jax
import jax.numpy as jnp
from jax import lax
from jax.experimental import pallas as pl
from jax.experimental.pallas import tpu as pltpu
from jax.experimental.pallas import tpu_sc as plsc

K = 512
D = 32
DA = 40
TW = 48
DECAY = 0.9
EPS = 1e-5

NC = 2
NS = 16
NW = NC * NS
CH = 1024


def _argmin_body(T, NB, x_ref, emb_ref, idx_ref, sx2_ref, embA_ref, acc_ref):
    i = pl.program_id(0)

    @pl.when(i == 0)
    def _():
        E = emb_ref[...]
        e2 = jnp.sum(E * E, axis=1, keepdims=True)
        embA_ref[...] = jnp.concatenate(
            [-2.0 * E, e2, jnp.zeros((K, DA - D - 1), jnp.float32)], axis=1)
        acc_ref[0] = 0.0

    x = x_ref[...]
    xa = jnp.concatenate([x, jnp.ones((T, DA - D), jnp.float32)], axis=1)
    dist2 = lax.dot_general(embA_ref[...], xa, (((1,), (1,)), ((), ())),
                            preferred_element_type=jnp.float32)
    mind = jnp.min(dist2, axis=0, keepdims=True)
    fiota = lax.broadcasted_iota(jnp.int32, (K, T), 0).astype(jnp.float32)
    cand = jnp.where(dist2 == mind, fiota, float(K))
    idxf = jnp.min(cand, axis=0, keepdims=True)
    idx_ref[...] = idxf[0, :].astype(jnp.int32)
    acc_ref[0] += jnp.sum(x * x)

    @pl.when(i == NB - 1)
    def _():
        sx2_ref[...] = jnp.reshape(acc_ref[0], (1, 1))


def _tc_argmin(flat, embedding, T=8192):
    N = flat.shape[0]
    NB = N // T
    idx, sx2 = pl.pallas_call(
        functools.partial(_argmin_body, T, NB),
        grid=(NB,),
        in_specs=[
            pl.BlockSpec((T, D), lambda i: (i, 0)),
            pl.BlockSpec((K, D), lambda i: (0, 0)),
        ],
        out_specs=[
            pl.BlockSpec((T,), lambda i: (i,)),
            pl.BlockSpec((1, 1), lambda i: (0, 0)),
        ],
        out_shape=[
            jax.ShapeDtypeStruct((N,), jnp.int32),
            jax.ShapeDtypeStruct((1, 1), jnp.float32),
        ],
        scratch_shapes=[
            pltpu.VMEM((K, DA), jnp.float32),
            pltpu.SMEM((1,), jnp.float32),
        ],
    )(flat, embedding)
    return idx, sx2


def _sc_body(NPW, xflat_hbm, idx_hbm, out_hbm, tab, tb2,
             xbuf0, ibuf0, xbuf1, ibuf1, sx0, si0, sx1, si1):
    c = lax.axis_index("c")
    s = lax.axis_index("s")
    wid = c * NS + s
    tok0 = wid * NPW

    zeros16 = jnp.zeros((16,), jnp.float32)
    iota16 = lax.broadcasted_iota(jnp.int32, (16,), 0)
    cnt16 = jnp.where(iota16 == 0, 1.0, 0.0).astype(jnp.float32)

    def zero_body(j, _):
        tab[pl.ds(j * 16, 16)] = zeros16
        tb2[pl.ds(j * 16, 16)] = zeros16
        return 0
    lax.fori_loop(0, K * TW // 16, zero_body, 0)

    bufs = [(xbuf0, ibuf0, sx0, si0), (xbuf1, ibuf1, sx1, si1)]
    NCH = NPW // CH

    def start(ci):
        xb, ib, sx, si = bufs[ci % 2]
        base = tok0 + ci * CH
        dx = pltpu.async_copy(xflat_hbm.at[pl.ds(base * D, CH * D)], xb, sx)
        di = pltpu.async_copy(idx_hbm.at[pl.ds(base, CH)], ib, si)
        return dx, di

    pend = start(0)
    for ci in range(NCH):
        xb, ib, sx, si = bufs[ci % 2]
        pend[0].wait()
        pend[1].wait()
        if ci + 1 < NCH:
            pend = start(ci + 1)

        def grp_body(g, _, xb=xb, ib=ib):
            iv = ib[pl.ds(g * 16, 16)]
            for j in range(16):
                t = g * 16 + j
                dst = tab if j % 2 == 0 else tb2
                o = iv[j] * TW
                xlo = xb[pl.ds(t * D, 16)]
                xhi = xb[pl.ds(t * D + 16, 16)]
                plsc.addupdate(dst.at[pl.ds(o, 16)], xlo)
                plsc.addupdate(dst.at[pl.ds(o + 16, 16)], xhi)
                plsc.addupdate(dst.at[pl.ds(o + 32, 16)], cnt16)
            return 0
        lax.fori_loop(0, CH // 16, grp_body, 0)

    def merge_body(j, _):
        tab[pl.ds(j * 16, 16)] += tb2[pl.ds(j * 16, 16)]
        return 0
    lax.fori_loop(0, K * TW // 16, merge_body, 0)
    pltpu.sync_copy(tab, out_hbm.at[pl.ds(wid * (K * TW), K * TW)])


def _sc_scatter(xflat, idx):
    N = idx.shape[0]
    NPW = N // NW
    mesh = plsc.VectorSubcoreMesh(core_axis_name="c", subcore_axis_name="s",
                                  num_cores=NC, num_subcores=NS)
    f = pl.kernel(
        functools.partial(_sc_body, NPW),
        out_type=jax.ShapeDtypeStruct((NW * K * TW,), jnp.float32),
        mesh=mesh,
        scratch_types=[
            pltpu.VMEM((K * TW,), jnp.float32),
            pltpu.VMEM((K * TW,), jnp.float32),
            pltpu.VMEM((CH * D,), jnp.float32),
            pltpu.VMEM((CH,), jnp.int32),
            pltpu.VMEM((CH * D,), jnp.float32),
            pltpu.VMEM((CH,), jnp.int32),
            pltpu.SemaphoreType.DMA,
            pltpu.SemaphoreType.DMA,
            pltpu.SemaphoreType.DMA,
            pltpu.SemaphoreType.DMA,
        ],
    )
    return f(xflat, idx).reshape(NW, K, TW)


def _final_body(N, part_ref, emaw_ref, cs_ref, sx2_ref, out_ref, acc_ref):
    i = pl.program_id(0)

    @pl.when(i == 0)
    def _():
        acc_ref[...] = jnp.zeros_like(acc_ref)

    acc_ref[...] += part_ref[0]

    @pl.when(i == NW - 1)
    def _():
        counts = acc_ref[:, D:D + 1]
        dw = acc_ref[:, 0:D]
        cs = cs_ref[...] * DECAY + (1.0 - DECAY) * counts
        n = jnp.sum(cs)
        csn = (cs + EPS) / (n + K * EPS) * n
        ema_w_new = emaw_ref[...] * DECAY + (1.0 - DECAY) * dw
        e_new = ema_w_new / csn
        s1 = jnp.sum(dw * e_new)
        s2 = jnp.sum(counts * jnp.sum(e_new * e_new, axis=1, keepdims=True))
        loss = (sx2_ref[0, 0] - 2.0 * s1 + s2) / (N * D)
        out_ref[...] = jnp.reshape(loss, (1, 1))


def _tc_final(parts, ema_w, cs, sx2, N):
    return pl.pallas_call(
        functools.partial(_final_body, N),
        grid=(NW,),
        in_specs=[
            pl.BlockSpec((1, K, TW), lambda i: (i, 0, 0)),
            pl.BlockSpec((K, D), lambda i: (0, 0)),
            pl.BlockSpec((K, 1), lambda i: (0, 0)),
            pl.BlockSpec((1, 1), lambda i: (0, 0)),
        ],
        out_specs=pl.BlockSpec((1, 1), lambda i: (0, 0)),
        out_shape=jax.ShapeDtypeStruct((1, 1), jnp.float32),
        scratch_shapes=[pltpu.VMEM((K, TW), jnp.float32)],
    )(parts, ema_w, cs, sx2)


def kernel(inputs, embedding, ema_w, ema_cluster_size):
    N = inputs.shape[0] * inputs.shape[1]
    flat = inputs.reshape(N, D)
    idx, sx2 = _tc_argmin(flat, embedding)
    parts = _sc_scatter(flat.reshape(N * D), idx)
    loss = _tc_final(parts, ema_w, ema_cluster_size.reshape(K, 1), sx2, N)
    return loss[0, 0]
